# baseline (speedup 1.0000x reference)
import functools

import jax
import jax.numpy as jnp
from jax import lax
from jax.experimental import pallas as pl
from jax.experimental.pallas import tpu as pltpu
from jax.experimental.pallas import tpu_sc as plsc

BATCH = 16
BEAM = 8
VOCAB = 32768
ENC_LEN = 2048
EOS_INDEX = 32767
MAX_ATTN_SHIFT = 60.0
EOS_THRESHOLD = 1.5
MINUS_INF = -1e20
SENT = -3.4e38
BIGI = 0x7FFFFFFF
NCH = 64
CHN = VOCAB // NCH
CHV = CHN // 16
ATV = ENC_LEN // 16


def _vmax(vs):
  return functools.reduce(jnp.maximum, vs)


def _vmin(vs):
  return functools.reduce(jnp.minimum, vs)


def _body(lp_hbm, attn_hbm, prev_hbm, seq_hbm,
          sc_out, tok_out, pred_out,
          row_big, attn_buf, prev_buf, seq_buf,
          cand_v, cand_i, mg_v, mg_i, st_v, st_t, st_p,
          vals_sh, idx_sh,
          sem_a, sem_r0, sem_r1):
  c = lax.axis_index("c")
  s = lax.axis_index("s")
  lane = lax.iota(jnp.int32, 16)
  sentv = jnp.full((16,), SENT, jnp.float32)
  bigv = jnp.full((16,), BIGI, jnp.int32)
  zerov = jnp.zeros((16,), jnp.int32)

  bb = s // 2
  h = s % 2
  r0 = c * 64 + s * 4

  pltpu.sync_copy(prev_hbm, prev_buf)
  pltpu.sync_copy(seq_hbm, seq_buf)

  def _extract_f(buf, idx):
    v = buf[pl.ds((idx // 16) * 16, 16)]
    return jnp.max(jnp.where(lane == idx % 16, v, SENT))

  for i in range(4):
    pltpu.async_copy(attn_hbm.at[r0 + i],
                     attn_buf.at[pl.ds(i * ENC_LEN, ENC_LEN)], sem_a)
  for i in range(4):
    pltpu.make_async_copy(attn_hbm.at[r0 + i],
                          attn_buf.at[pl.ds(i * ENC_LEN, ENC_LEN)],
                          sem_a).wait()

  def attn_row(i2, cvec):
    ab = i2 * ENC_LEN

    def attn_step(jj, carry):
      lm, aj = carry
      for u in range(8):
        v = attn_buf[pl.ds(ab + (jj * 8 + u) * 16, 16)]
        upd = v > lm
        lm = jnp.where(upd, v, lm)
        aj = jnp.where(upd, jj * 8 + u, aj)
      return lm, aj

    lm, aj = lax.fori_loop(0, ATV // 8, attn_step, (sentv, zerov))
    am = jnp.max(lm)
    peak = jnp.min(jnp.where(lm == am, aj * 16 + lane, BIGI))
    pv = _extract_f(prev_buf, r0 + i2)
    cnd = peak.astype(jnp.float32) < pv + MAX_ATTN_SHIFT
    return jnp.where(lane == i2, cnd.astype(jnp.int32), cvec)

  cvec = lax.fori_loop(0, 4, attn_row, zerov)

  def _cond(i):
    return jnp.max(jnp.where(lane == i, cvec, 0)) > 0

  @pl.when(_cond(0))
  def _():
    pltpu.async_copy(lp_hbm.at[r0], row_big.at[pl.ds(0, VOCAB)], sem_r0)

  @pl.when(_cond(1))
  def _():
    pltpu.async_copy(lp_hbm.at[r0 + 1],
                     row_big.at[pl.ds(VOCAB, VOCAB)], sem_r1)

  def process_row(base0, i_dyn):
    sv = _extract_f(seq_buf, r0 + i_dyn)
    rowinb = h * 4 + i_dyn

    lv = row_big[pl.ds(base0 + VOCAB - 16, 16)]
    e = jnp.max(jnp.where(lane == 15, lv, SENT))
    row_big[pl.ds(base0 + VOCAB - 16, 16)] = jnp.where(lane == 15, SENT, lv)

    cms_l = []
    for g in range(4):
      idx0 = base0 + g * 16 * CHN + lane * CHN

      def gstep(tt, carry, idx0=idx0):
        m, idx = carry
        for u in range(8):
          v = plsc.load_gather(row_big, [idx])
          m = jnp.maximum(m, v)
          idx = idx + 1
        return m, idx

      m, _ = lax.fori_loop(0, CHN // 8, gstep, (sentv, idx0))
      cms_l.append(m)
    cms = tuple(cms_l)
    max_non_eos = jnp.max(_vmax(cms))
    e_new = jnp.where(e < EOS_THRESHOLD * max_non_eos,
                      jnp.float32(MINUS_INF), e)
    lv2 = row_big[pl.ds(base0 + VOCAB - 16, 16)]
    row_big[pl.ds(base0 + VOCAB - 16, 16)] = jnp.where(lane == 15, e_new, lv2)
    cms = (cms[0], cms[1], cms[2],
           jnp.where(lane == 15, jnp.maximum(cms[3], e_new), cms[3]))

    def topk_step(t, carry):
      cm0, cm1, cm2, cm3, v16, i16 = carry
      cmst = (cm0, cm1, cm2, cm3)
      mval = jnp.max(_vmax(cmst))
      cstar = jnp.min(_vmin(tuple(
          jnp.where(cmst[k] == mval, lane + 16 * k, BIGI)
          for k in range(4))))
      base = base0 + cstar * CHN

      def scan_step(jj, carry2):
        slm, sl2, saj = carry2
        for u in range(8):
          v = row_big[pl.ds(base + (jj * 8 + u) * 16, 16)]
          upd = v > slm
          sl2 = jnp.maximum(sl2, jnp.where(upd, slm, v))
          slm = jnp.where(upd, v, slm)
          saj = jnp.where(upd, jj * 8 + u, saj)
        return slm, sl2, saj

      slm, sl2, saj = lax.fori_loop(0, CHV // 8, scan_step,
                                    (sentv, sentv, zerov))
      pos = jnp.min(jnp.where(slm == mval, saj * 16 + lane, BIGI))
      absi = (base - base0) + pos
      kb = base + (pos // 16) * 16
      kl = pos % 16
      kv = row_big[pl.ds(kb, 16)]
      row_big[pl.ds(kb, 16)] = jnp.where(lane == kl, SENT, kv)
      nm = jnp.max(jnp.where(lane == kl, sl2, slm))
      new_cms = tuple(
          jnp.where(lane + 16 * k == cstar, nm, cmst[k]) for k in range(4))
      v16 = jnp.where(lane == t, mval, v16)
      i16 = jnp.where(lane == t, rowinb * VOCAB + absi, i16)
      return (*new_cms, v16, i16)

    out = lax.fori_loop(0, 8, topk_step, (*cms, sentv, bigv))
    cand_v[pl.ds(i_dyn * 16, 16)] = out[4] + sv
    cand_i[pl.ds(i_dyn * 16, 16)] = out[5]

  def trivial_row(i_dyn):
    sv = _extract_f(seq_buf, r0 + i_dyn)
    rowinb = h * 4 + i_dyn
    fv = jnp.float32(MINUS_INF) + sv
    cand_v[pl.ds(i_dyn * 16, 16)] = jnp.where(lane < 8, fv, SENT)
    cand_i[pl.ds(i_dyn * 16, 16)] = jnp.where(lane < 8,
                                              rowinb * VOCAB + lane, BIGI)

  def row_step(i_dyn, _):
    p0 = i_dyn % 2 == 0
    cnd = _cond(i_dyn)

    @pl.when(jnp.logical_and(cnd, p0))
    def _():
      pltpu.make_async_copy(lp_hbm.at[r0 + i_dyn],
                            row_big.at[pl.ds(0, VOCAB)], sem_r0).wait()

    @pl.when(jnp.logical_and(cnd, jnp.logical_not(p0)))
    def _():
      pltpu.make_async_copy(lp_hbm.at[r0 + i_dyn],
                            row_big.at[pl.ds(VOCAB, VOCAB)], sem_r1).wait()

    @pl.when(cnd)
    def _():
      process_row((i_dyn % 2) * VOCAB, i_dyn)

    @pl.when(jnp.logical_not(cnd))
    def _():
      trivial_row(i_dyn)

    nx = jnp.logical_and(i_dyn < 2, _cond(i_dyn + 2))

    @pl.when(jnp.logical_and(nx, p0))
    def _():
      pltpu.async_copy(lp_hbm.at[r0 + i_dyn + 2],
                       row_big.at[pl.ds(0, VOCAB)], sem_r0)

    @pl.when(jnp.logical_and(nx, jnp.logical_not(p0)))
    def _():
      pltpu.async_copy(lp_hbm.at[r0 + i_dyn + 2],
                       row_big.at[pl.ds(VOCAB, VOCAB)], sem_r1)
    return 0

  lax.fori_loop(0, 4, row_step, 0)

  pltpu.sync_copy(cand_v, vals_sh.at[bb, pl.ds(h * 64, 64)])
  pltpu.sync_copy(cand_i, idx_sh.at[bb, pl.ds(h * 64, 64)])
  plsc.subcore_barrier()

  @pl.when(s < 8)
  def _():
    pltpu.sync_copy(vals_sh.at[s], mg_v)
    pltpu.sync_copy(idx_sh.at[s], mg_i)
    bg = c * 8 + s

    def merge_step(t, carry):
      (m0, m1, m2, m3, m4, m5, m6, m7, sc16, tk16, pd16) = carry
      mvs = (m0, m1, m2, m3, m4, m5, m6, m7)
      mis = [mg_i[pl.ds(kk * 16, 16)] for kk in range(8)]
      mval = jnp.max(_vmax(mvs))
      wi = jnp.min(_vmin([jnp.where(mvs[kk] == mval, mis[kk], BIGI)
                          for kk in range(8)]))
      mvs = tuple(jnp.where(mis[kk] == wi, SENT, mvs[kk]) for kk in range(8))
      sc16 = jnp.where(lane == t, mval, sc16)
      tk16 = jnp.where(lane == t, wi & (VOCAB - 1), tk16)
      pd16 = jnp.where(lane == t, (wi >> 15) + bg * 8, pd16)
      return (*mvs, sc16, tk16, pd16)

    init = tuple(mg_v[pl.ds(kk * 16, 16)] for kk in range(8))
    out = lax.fori_loop(0, 8, merge_step, (*init, sentv, zerov, zerov))
    st_v[...] = out[8]
    st_t[...] = out[9]
    st_p[...] = out[10]
    pltpu.sync_copy(st_v.at[pl.ds(0, 8)], sc_out.at[pl.ds(bg * 8, 8)])
    pltpu.sync_copy(st_t.at[pl.ds(0, 8)], tok_out.at[pl.ds(bg * 8, 8)])
    pltpu.sync_copy(st_p.at[pl.ds(0, 8)], pred_out.at[pl.ds(bg * 8, 8)])


@jax.jit
def _run(log_probs, attn, prev_attn_peak, sequence_scores):
  mesh = plsc.VectorSubcoreMesh(core_axis_name="c", subcore_axis_name="s",
                                num_cores=2, num_subcores=16)
  f = pl.kernel(
      _body,
      out_type=(
          jax.ShapeDtypeStruct((BATCH * BEAM,), jnp.float32),
          jax.ShapeDtypeStruct((BATCH * BEAM,), jnp.int32),
          jax.ShapeDtypeStruct((BATCH * BEAM,), jnp.int32),
      ),
      mesh=mesh,
      compiler_params=pltpu.CompilerParams(needs_layout_passes=False),
      scratch_types=[
          pltpu.VMEM((2 * VOCAB,), jnp.float32),
          pltpu.VMEM((4 * ENC_LEN,), jnp.float32),
          pltpu.VMEM((128,), jnp.float32),
          pltpu.VMEM((128,), jnp.float32),
          pltpu.VMEM((64,), jnp.float32),
          pltpu.VMEM((64,), jnp.int32),
          pltpu.VMEM((128,), jnp.float32),
          pltpu.VMEM((128,), jnp.int32),
          pltpu.VMEM((16,), jnp.float32),
          pltpu.VMEM((16,), jnp.int32),
          pltpu.VMEM((16,), jnp.int32),
          pltpu.VMEM_SHARED((8, 128), jnp.float32),
          pltpu.VMEM_SHARED((8, 128), jnp.int32),
          pltpu.SemaphoreType.DMA,
          pltpu.SemaphoreType.DMA,
          pltpu.SemaphoreType.DMA,
      ],
  )
  sc, tok, pred = f(log_probs, attn, prev_attn_peak, sequence_scores)
  return (sc.reshape(BATCH, BEAM), tok.reshape(BATCH, BEAM),
          pred.reshape(BATCH, BEAM))


def kernel(log_probs, attn, prev_attn_peak, sequence_scores):
  return _run(log_probs, attn, prev_attn_peak, sequence_scores)


# speedup vs baseline: 2.2086x; 2.2086x over previous
import functools

import jax
import jax.numpy as jnp
from jax import lax
from jax.experimental import pallas as pl
from jax.experimental.pallas import tpu as pltpu
from jax.experimental.pallas import tpu_sc as plsc

BATCH = 16
BEAM = 8
VOCAB = 32768
ENC_LEN = 2048
EOS_INDEX = 32767
MAX_ATTN_SHIFT = 60.0
EOS_THRESHOLD = 1.5
MINUS_INF = -1e20
SENT = -3.4e38
BIGI = 0x7FFFFFFF
NCH = 64
CHN = VOCAB // NCH
CHV = CHN // 16
ATV = ENC_LEN // 16


def _vmax(vs):
  return functools.reduce(jnp.maximum, vs)


def _vmin(vs):
  return functools.reduce(jnp.minimum, vs)


def _body(lp_hbm, attn_hbm, prev_hbm, seq_hbm,
          sc_out, tok_out, pred_out,
          row_big, attn_buf, prev_buf, seq_buf,
          cbuf, ct_v, ct_i, mg_v, mg_i, st_v, st_t, st_p,
          vals_sh, idx_sh, cond_sh,
          sem_a, sem_r0, sem_r1):
  c = lax.axis_index("c")
  s = lax.axis_index("s")
  lane = lax.iota(jnp.int32, 16)
  sentv = jnp.full((16,), SENT, jnp.float32)
  bigv = jnp.full((16,), BIGI, jnp.int32)
  zerov = jnp.zeros((16,), jnp.int32)

  bb = s // 2
  h = s % 2
  r0 = c * 64 + s * 4

  pltpu.sync_copy(prev_hbm, prev_buf)
  pltpu.sync_copy(seq_hbm, seq_buf)

  def _extract_f(buf, idx):
    v = buf[pl.ds((idx // 16) * 16, 16)]
    return jnp.max(jnp.where(lane == idx % 16, v, SENT))

  for i in range(4):
    pltpu.async_copy(attn_hbm.at[r0 + i],
                     attn_buf.at[pl.ds(i * ENC_LEN, ENC_LEN)], sem_a)
  for i in range(4):
    pltpu.make_async_copy(attn_hbm.at[r0 + i],
                          attn_buf.at[pl.ds(i * ENC_LEN, ENC_LEN)],
                          sem_a).wait()

  def attn_row(i2, cvec):
    ab = i2 * ENC_LEN

    def attn_step(jj, carry):
      lm, aj = carry
      for u in range(8):
        v = attn_buf[pl.ds(ab + (jj * 8 + u) * 16, 16)]
        upd = v > lm
        lm = jnp.where(upd, v, lm)
        aj = jnp.where(upd, jj * 8 + u, aj)
      return lm, aj

    lm, aj = lax.fori_loop(0, ATV // 8, attn_step, (sentv, zerov))
    am = jnp.max(lm)
    peak = jnp.min(jnp.where(lm == am, aj * 16 + lane, BIGI))
    pv = _extract_f(prev_buf, r0 + i2)
    cnd = peak.astype(jnp.float32) < pv + MAX_ATTN_SHIFT
    return jnp.where(lane == i2, cnd.astype(jnp.int32), cvec)

  cvec = lax.fori_loop(0, 4, attn_row, zerov)

  def _cond(i):
    return jnp.max(jnp.where(lane == i, cvec, 0)) > 0

  ct_i[...] = cvec
  pltpu.sync_copy(ct_i, cond_sh.at[pl.ds(s * 16, 16)])
  plsc.subcore_barrier()
  pltpu.sync_copy(cond_sh, cbuf)
  cvs = []
  qs = []
  offs = 0
  for k in range(4):
    gidx = (4 * k + lane // 4) * 16 + (lane % 4)
    cvk = plsc.load_gather(cbuf, [gidx])
    qk = plsc.cumsum(cvk) - cvk + offs
    offs = offs + jnp.sum(cvk)
    cvs.append(cvk)
    qs.append(qk)
  ms = []
  for t in range(4):
    cand = [jnp.where(
        jnp.logical_and(
            cvs[k] == 1,
            jnp.logical_and(qs[k] % 16 == s, qs[k] // 16 == t)),
        16 * k + lane, BIGI) for k in range(4)]
    ms.append(jnp.min(_vmin(cand)))
  myrows = jnp.where(lane == 0, ms[0],
                     jnp.where(lane == 1, ms[1],
                               jnp.where(lane == 2, ms[2],
                                         jnp.where(lane == 3, ms[3], BIGI))))

  def _mrow(t):
    return jnp.min(jnp.where(lane == t, myrows, BIGI))

  for i in range(4):
    @pl.when(jnp.logical_not(_cond(i)))
    def _(i=i):
      sv = _extract_f(seq_buf, r0 + i)
      rowinb = h * 4 + i
      fv = jnp.float32(MINUS_INF) + sv
      ct_v[...] = jnp.where(lane < 8, fv, SENT)
      ct_i[...] = jnp.where(lane < 8, rowinb * VOCAB + lane, BIGI)
      pltpu.sync_copy(ct_v, vals_sh.at[bb, pl.ds(rowinb * 16, 16)])
      pltpu.sync_copy(ct_i, idx_sh.at[bb, pl.ds(rowinb * 16, 16)])

  @pl.when(ms[0] < BIGI)
  def _():
    pltpu.async_copy(lp_hbm.at[c * 64 + ms[0]],
                     row_big.at[pl.ds(0, VOCAB)], sem_r0)

  @pl.when(ms[1] < BIGI)
  def _():
    pltpu.async_copy(lp_hbm.at[c * 64 + ms[1]],
                     row_big.at[pl.ds(VOCAB, VOCAB)], sem_r1)

  def process_row(base0, r_local):
    sv = _extract_f(seq_buf, c * 64 + r_local)
    rowinb = r_local % 8

    lv = row_big[pl.ds(base0 + VOCAB - 16, 16)]
    e = jnp.max(jnp.where(lane == 15, lv, SENT))
    row_big[pl.ds(base0 + VOCAB - 16, 16)] = jnp.where(lane == 15, SENT, lv)

    def chunk_pass(c64, cms):
      m = sentv
      for j in range(CHV):
        m = jnp.maximum(m, row_big[pl.ds(base0 + c64 * CHN + j * 16, 16)])
      mx = jnp.max(m)
      return tuple(
          jnp.where(lane + 16 * k == c64, mx, cms[k]) for k in range(4))

    cms = lax.fori_loop(0, NCH, chunk_pass, (sentv,) * 4)
    max_non_eos = jnp.max(_vmax(cms))
    e_new = jnp.where(e < EOS_THRESHOLD * max_non_eos,
                      jnp.float32(MINUS_INF), e)
    lv2 = row_big[pl.ds(base0 + VOCAB - 16, 16)]
    row_big[pl.ds(base0 + VOCAB - 16, 16)] = jnp.where(lane == 15, e_new, lv2)
    cms = (cms[0], cms[1], cms[2],
           jnp.where(lane == 15, jnp.maximum(cms[3], e_new), cms[3]))

    def topk_step(t, carry):
      cm0, cm1, cm2, cm3, v16, i16 = carry
      cmst = (cm0, cm1, cm2, cm3)
      mval = jnp.max(_vmax(cmst))
      cstar = jnp.min(_vmin(tuple(
          jnp.where(cmst[k] == mval, lane + 16 * k, BIGI)
          for k in range(4))))
      base = base0 + cstar * CHN

      def scan_step(jj, carry2):
        slm, sl2, saj = carry2
        for u in range(8):
          v = row_big[pl.ds(base + (jj * 8 + u) * 16, 16)]
          upd = v > slm
          sl2 = jnp.maximum(sl2, jnp.where(upd, slm, v))
          slm = jnp.where(upd, v, slm)
          saj = jnp.where(upd, jj * 8 + u, saj)
        return slm, sl2, saj

      slm, sl2, saj = lax.fori_loop(0, CHV // 8, scan_step,
                                    (sentv, sentv, zerov))
      pos = jnp.min(jnp.where(slm == mval, saj * 16 + lane, BIGI))
      absi = (base - base0) + pos
      kb = base + (pos // 16) * 16
      kl = pos % 16
      kv = row_big[pl.ds(kb, 16)]
      row_big[pl.ds(kb, 16)] = jnp.where(lane == kl, SENT, kv)
      nm = jnp.max(jnp.where(lane == kl, sl2, slm))
      new_cms = tuple(
          jnp.where(lane + 16 * k == cstar, nm, cmst[k]) for k in range(4))
      v16 = jnp.where(lane == t, mval, v16)
      i16 = jnp.where(lane == t, rowinb * VOCAB + absi, i16)
      return (*new_cms, v16, i16)

    out = lax.fori_loop(0, 8, topk_step, (*cms, sentv, bigv))
    ct_v[...] = out[4] + sv
    ct_i[...] = out[5]
    pltpu.sync_copy(ct_v, vals_sh.at[r_local // 8, pl.ds(rowinb * 16, 16)])
    pltpu.sync_copy(ct_i, idx_sh.at[r_local // 8, pl.ds(rowinb * 16, 16)])

  def live_step(t, _):
    mt = _mrow(t)
    live = mt < BIGI
    p0 = t % 2 == 0

    @pl.when(jnp.logical_and(live, p0))
    def _():
      pltpu.make_async_copy(lp_hbm.at[c * 64 + mt],
                            row_big.at[pl.ds(0, VOCAB)], sem_r0).wait()

    @pl.when(jnp.logical_and(live, jnp.logical_not(p0)))
    def _():
      pltpu.make_async_copy(lp_hbm.at[c * 64 + mt],
                            row_big.at[pl.ds(VOCAB, VOCAB)], sem_r1).wait()

    @pl.when(live)
    def _():
      process_row((t % 2) * VOCAB, mt)

    mn = _mrow(t + 2)
    nx = mn < BIGI

    @pl.when(jnp.logical_and(nx, p0))
    def _():
      pltpu.async_copy(lp_hbm.at[c * 64 + mn],
                       row_big.at[pl.ds(0, VOCAB)], sem_r0)

    @pl.when(jnp.logical_and(nx, jnp.logical_not(p0)))
    def _():
      pltpu.async_copy(lp_hbm.at[c * 64 + mn],
                       row_big.at[pl.ds(VOCAB, VOCAB)], sem_r1)
    return 0

  lax.fori_loop(0, 4, live_step, 0)

  plsc.subcore_barrier()

  @pl.when(s < 8)
  def _():
    pltpu.sync_copy(vals_sh.at[s], mg_v)
    pltpu.sync_copy(idx_sh.at[s], mg_i)
    bg = c * 8 + s

    def merge_step(t, carry):
      (m0, m1, m2, m3, m4, m5, m6, m7, sc16, tk16, pd16) = carry
      mvs = (m0, m1, m2, m3, m4, m5, m6, m7)
      mis = [mg_i[pl.ds(kk * 16, 16)] for kk in range(8)]
      mval = jnp.max(_vmax(mvs))
      wi = jnp.min(_vmin([jnp.where(mvs[kk] == mval, mis[kk], BIGI)
                          for kk in range(8)]))
      mvs = tuple(jnp.where(mis[kk] == wi, SENT, mvs[kk]) for kk in range(8))
      sc16 = jnp.where(lane == t, mval, sc16)
      tk16 = jnp.where(lane == t, wi & (VOCAB - 1), tk16)
      pd16 = jnp.where(lane == t, (wi >> 15) + bg * 8, pd16)
      return (*mvs, sc16, tk16, pd16)

    init = tuple(mg_v[pl.ds(kk * 16, 16)] for kk in range(8))
    out = lax.fori_loop(0, 8, merge_step, (*init, sentv, zerov, zerov))
    st_v[...] = out[8]
    st_t[...] = out[9]
    st_p[...] = out[10]
    pltpu.sync_copy(st_v.at[pl.ds(0, 8)], sc_out.at[pl.ds(bg * 8, 8)])
    pltpu.sync_copy(st_t.at[pl.ds(0, 8)], tok_out.at[pl.ds(bg * 8, 8)])
    pltpu.sync_copy(st_p.at[pl.ds(0, 8)], pred_out.at[pl.ds(bg * 8, 8)])


@jax.jit
def _run(log_probs, attn, prev_attn_peak, sequence_scores):
  mesh = plsc.VectorSubcoreMesh(core_axis_name="c", subcore_axis_name="s",
                                num_cores=2, num_subcores=16)
  f = pl.kernel(
      _body,
      out_type=(
          jax.ShapeDtypeStruct((BATCH * BEAM,), jnp.float32),
          jax.ShapeDtypeStruct((BATCH * BEAM,), jnp.int32),
          jax.ShapeDtypeStruct((BATCH * BEAM,), jnp.int32),
      ),
      mesh=mesh,
      compiler_params=pltpu.CompilerParams(needs_layout_passes=False),
      scratch_types=[
          pltpu.VMEM((2 * VOCAB,), jnp.float32),
          pltpu.VMEM((4 * ENC_LEN,), jnp.float32),
          pltpu.VMEM((128,), jnp.float32),
          pltpu.VMEM((128,), jnp.float32),
          pltpu.VMEM((256,), jnp.int32),
          pltpu.VMEM((16,), jnp.float32),
          pltpu.VMEM((16,), jnp.int32),
          pltpu.VMEM((128,), jnp.float32),
          pltpu.VMEM((128,), jnp.int32),
          pltpu.VMEM((16,), jnp.float32),
          pltpu.VMEM((16,), jnp.int32),
          pltpu.VMEM((16,), jnp.int32),
          pltpu.VMEM_SHARED((8, 128), jnp.float32),
          pltpu.VMEM_SHARED((8, 128), jnp.int32),
          pltpu.VMEM_SHARED((256,), jnp.int32),
          pltpu.SemaphoreType.DMA,
          pltpu.SemaphoreType.DMA,
          pltpu.SemaphoreType.DMA,
      ],
  )
  sc, tok, pred = f(log_probs, attn, prev_attn_peak, sequence_scores)
  return (sc.reshape(BATCH, BEAM), tok.reshape(BATCH, BEAM),
          pred.reshape(BATCH, BEAM))


def kernel(log_probs, attn, prev_attn_peak, sequence_scores):
  return _run(log_probs, attn, prev_attn_peak, sequence_scores)
